# Initial kernel scaffold; baseline (speedup 1.0000x reference)
import jax
import jax.numpy as jnp
from jax.experimental import pallas as pl


def kernel(x, table, pos_embedding):
    raise NotImplementedError("write your pallas kernel here")


# speedup vs baseline: 2.4890x; 2.4890x over previous
import functools

import jax
import jax.numpy as jnp
from jax import lax
from jax.experimental import pallas as pl
from jax.experimental.pallas import tpu as pltpu
from jax.experimental.pallas import tpu_sc as plsc

NC = 2
NS = 16
NW = NC * NS
L = 16


@functools.partial(jax.jit, static_argnames=("B", "S", "D"))
def _emb_call(x_flat, table, pos, *, B, S, D):
    rows_per_w = B // NW
    n_a = 128
    n_b = S - n_a
    mesh = plsc.VectorSubcoreMesh(core_axis_name="c", subcore_axis_name="s")

    @functools.partial(
        pl.kernel,
        mesh=mesh,
        compiler_params=pltpu.CompilerParams(use_tc_tiling_on_sc=False),
        out_type=jax.ShapeDtypeStruct((B * S, D), jnp.float32),
        scratch_types=[
            pltpu.VMEM((n_a,), jnp.int32),
            pltpu.VMEM((n_b,), jnp.int32),
            pltpu.VMEM((S, D), jnp.float32),
            pltpu.VMEM((S, D), jnp.float32),
            pltpu.SemaphoreType.DMA,
        ],
    )
    def k(x_hbm, table_hbm, pos_hbm, out_hbm, idx_a, idx_b, buf_v, pos_v, sem):
        wid = lax.axis_index("s") * NC + lax.axis_index("c")
        pltpu.sync_copy(pos_hbm, pos_v)

        def row_body(r, carry):
            base = (wid * rows_per_w + r) * S
            pltpu.sync_copy(x_hbm.at[pl.ds(base, n_a)], idx_a)
            pltpu.sync_copy(x_hbm.at[pl.ds(base + n_a, n_b)], idx_b)
            cp_a = pltpu.async_copy(table_hbm.at[idx_a], buf_v.at[pl.ds(0, n_a)], sem)
            cp_b = pltpu.async_copy(table_hbm.at[idx_b], buf_v.at[pl.ds(n_a, n_b)], sem)
            cp_a.wait()
            cp_b.wait()

            def add_body(i, c):
                for j in range(D // L):
                    sl = pl.ds(j * L, L)
                    buf_v[i, sl] = buf_v[i, sl] + pos_v[i, sl]
                return c

            lax.fori_loop(0, S, add_body, 0)
            pltpu.sync_copy(buf_v, out_hbm.at[pl.ds(base, S)])
            return carry

        lax.fori_loop(0, rows_per_w, row_body, 0)

    return k(x_flat, table, pos)


def kernel(x, table, pos_embedding):
    B, S = x.shape
    D = table.shape[1]
    x_flat = x.reshape(-1).astype(jnp.int32)
    pos = pos_embedding[:S].astype(jnp.float32)
    out = _emb_call(x_flat, table.astype(jnp.float32), pos, B=B, S=S, D=D)
    return out.reshape(B, S, D)


# speedup vs baseline: 3.2304x; 1.2979x over previous
import functools

import jax
import jax.numpy as jnp
from jax import lax
from jax.experimental import pallas as pl
from jax.experimental.pallas import tpu as pltpu
from jax.experimental.pallas import tpu_sc as plsc

NC = 2
NS = 16
NW = NC * NS
L = 16
NB = 4


@functools.partial(jax.jit, static_argnames=("B", "S", "D"))
def _emb_call(x_flat, table, pos, *, B, S, D):
    rows_per_w = B // NW
    n_a = 128
    n_b = S - n_a
    mesh = plsc.VectorSubcoreMesh(core_axis_name="c", subcore_axis_name="s")

    @functools.partial(
        pl.kernel,
        mesh=mesh,
        compiler_params=pltpu.CompilerParams(use_tc_tiling_on_sc=False),
        out_type=jax.ShapeDtypeStruct((B * S, D), jnp.float32),
        scratch_types=[
            pltpu.VMEM((rows_per_w * S,), jnp.int32),
            pltpu.VMEM((S, D), jnp.float32),
            [pltpu.VMEM((S, D), jnp.float32) for _ in range(NB)],
            [pltpu.SemaphoreType.DMA for _ in range(NB)],
            [pltpu.SemaphoreType.DMA for _ in range(NB)],
        ],
    )
    def k(x_hbm, table_hbm, pos_hbm, out_hbm, idx_v, pos_v, bufs, gsems, osems):
        wid = lax.axis_index("s") * NC + lax.axis_index("c")
        pltpu.sync_copy(pos_hbm, pos_v)
        pltpu.sync_copy(x_hbm.at[pl.ds(wid * (rows_per_w * S), rows_per_w * S)], idx_v)

        def start_gather(r):
            b = r % NB
            da = pltpu.async_copy(
                table_hbm.at[idx_v.at[pl.ds(r * S, n_a)]],
                bufs[b].at[pl.ds(0, n_a)], gsems[b])
            db = pltpu.async_copy(
                table_hbm.at[idx_v.at[pl.ds(r * S + n_a, n_b)]],
                bufs[b].at[pl.ds(n_a, n_b)], gsems[b])
            return (da, db)

        def start_out(r):
            b = r % NB
            return pltpu.async_copy(
                bufs[b], out_hbm.at[pl.ds((wid * rows_per_w + r) * S, S)], osems[b])

        gd = [None] * rows_per_w
        od = [None] * rows_per_w
        gd[0] = start_gather(0)
        gd[1] = start_gather(1)
        for r in range(rows_per_w):
            b = r % NB
            gd[r][0].wait()
            gd[r][1].wait()
            if r >= 2:
                od[r - 2].wait()
            if r + 2 < rows_per_w:
                gd[r + 2] = start_gather(r + 2)

            buf = bufs[b]

            def add_body(i, c):
                for j in range(D // L):
                    sl = pl.ds(j * L, L)
                    buf[i, sl] = buf[i, sl] + pos_v[i, sl]
                return c

            lax.fori_loop(0, S, add_body, 0)
            od[r] = start_out(r)
        od[rows_per_w - 2].wait()
        od[rows_per_w - 1].wait()

    return k(x_flat, table, pos)


def kernel(x, table, pos_embedding):
    B, S = x.shape
    D = table.shape[1]
    x_flat = x.reshape(-1).astype(jnp.int32)
    pos = pos_embedding[:S].astype(jnp.float32)
    out = _emb_call(x_flat, table.astype(jnp.float32), pos, B=B, S=S, D=D)
    return out.reshape(B, S, D)
